# baseline (speedup 1.0000x reference)
import functools

import jax
import jax.numpy as jnp
from jax import lax
from jax.experimental import pallas as pl
from jax.experimental.pallas import tpu as pltpu
from jax.experimental.pallas import tpu_sc as plsc

N = 100000
E = 1600000
G = 512
IN = 50
H = 64
OUT = 4

NC = 2
NS = 16
NW = NC * NS

Q = 25088
N_PAD = 4 * Q
ACC_ROWS = Q + 128
K = 128
KA = 64
ET = 50176
E_PAD = ET * NW
NCHUNK = ET // K

GP = 544
BN_PAD = N_PAD + 16
NODE_CH = 64
NODE_T = N_PAD // NW
NODE_NCH = NODE_T // NODE_CH

_mesh = plsc.VectorSubcoreMesh(
    core_axis_name="c", subcore_axis_name="s", num_cores=NC, num_subcores=NS)
_sc_params = pltpu.CompilerParams(use_tc_tiling_on_sc=False,
                                  needs_layout_passes=False)


def _zero_rows(rows, nrows):
  zero = jnp.zeros((16,), jnp.float32)

  def body(i, _):
    r = i // 4
    l = i % 4
    rows[r, pl.ds(l * 16, 16)] = zero
    return 0

  lax.fori_loop(0, nrows * 4, body, 0)


def _fill_ones(rows, nrows):
  one = jnp.ones((16,), jnp.float32)

  def body(i, _):
    r = i // 4
    l = i % 4
    rows[r, pl.ds(l * 16, 16)] = one
    return 0

  lax.fori_loop(0, nrows * 4, body, 0)


NB = 4
SUP = NB * KA

CAP = 14336
CH_BIN = 1792
NCH_BIN = ET // CH_BIN


def _bin_body(srcp, dstp, bsrc, bdst, bcnt,
              stage_s, stage_d, sbuf, dbuf, cntv, isem0, isem1):
  c = lax.axis_index("c")
  s = lax.axis_index("s")
  w = c * NS + s
  lane = lax.broadcasted_iota(jnp.int32, (16,), 0)

  def issue_idx(t, pb):
    eb = w * ET + t * CH_BIN
    pltpu.async_copy(srcp.at[pl.ds(eb, CH_BIN)], sbuf.at[pb], isem0)
    pltpu.async_copy(dstp.at[pl.ds(eb, CH_BIN)], dbuf.at[pb], isem1)

  def drain_idx():
    pltpu.make_async_copy(srcp.at[pl.ds(0, CH_BIN)], sbuf.at[0], isem0).wait()
    pltpu.make_async_copy(dstp.at[pl.ds(0, CH_BIN)], dbuf.at[0], isem1).wait()

  issue_idx(0, 0)

  def chunk(t, cnts):
    pb = t % 2
    drain_idx()

    @pl.when(t < NCH_BIN - 1)
    def _():
      issue_idx(t + 1, 1 - pb)

    def grp(g, cnts):
      d = dbuf[pb, pl.ds(g * 16, 16)]
      sv = sbuf[pb, pl.ds(g * 16, 16)]
      qv = ((d >= Q).astype(jnp.int32) + (d >= 2 * Q).astype(jnp.int32)
            + (d >= 3 * Q).astype(jnp.int32))
      rel = d - qv * Q
      new = []
      for b in range(4):
        cb = cnts[b]
        keep = (qv == b) & (rel < Q)
        mi = keep.astype(jnp.int32)
        m = keep & (cb < CAP - 16)
        pos = cb + plsc.cumsum(mi) - mi
        plsc.store_scatter(stage_s.at[b], [pos], sv, mask=m)
        plsc.store_scatter(stage_d.at[b], [pos], rel, mask=m)
        new.append(cb + jnp.sum(mi))
      return tuple(new)

    return lax.fori_loop(0, CH_BIN // 16, grp, cnts)

  cnts = lax.fori_loop(0, NCH_BIN, chunk, (0, 0, 0, 0))

  nchs = []
  for b in range(4):
    cb = jnp.minimum(cnts[b], CAP - SUP)
    target = ((cb + SUP - 1) // SUP) * SUP
    npadg = (target - cb + 15) // 16

    def padg(i, _, b=b, cb=cb):
      vsrc = (w * 1237 + i * 577 + lane * 61) % N
      vdst = Q + ((i * 16 + lane) & 127)
      stage_s[b, pl.ds(cb + i * 16, 16)] = vsrc
      stage_d[b, pl.ds(cb + i * 16, 16)] = vdst
      return 0

    lax.fori_loop(0, npadg, padg, 0)
    nchs.append(target // SUP)
    nfl = (target + 511) // 512

    def flush(j, _, b=b):
      pltpu.sync_copy(stage_s.at[b, pl.ds(j * 512, 512)],
                      bsrc.at[w, b, pl.ds(j * 512, 512)])
      pltpu.sync_copy(stage_d.at[b, pl.ds(j * 512, 512)],
                      bdst.at[w, b, pl.ds(j * 512, 512)])
      return 0

    lax.fori_loop(0, nfl, flush, 0)

  cvec = jnp.where(lane == 0, nchs[0],
                   jnp.where(lane == 1, nchs[1],
                             jnp.where(lane == 2, nchs[2],
                                       jnp.where(lane == 3, nchs[3], 0))))
  cntv[pl.ds(0, 16)] = cvec
  pltpu.sync_copy(cntv, bcnt.at[w])


_bin = pl.kernel(
    _bin_body,
    out_type=[
        jax.ShapeDtypeStruct((NW, 4, CAP), jnp.int32),
        jax.ShapeDtypeStruct((NW, 4, CAP), jnp.int32),
        jax.ShapeDtypeStruct((NW, 16), jnp.int32),
    ],
    mesh=_mesh,
    compiler_params=_sc_params,
    scratch_types=[
        pltpu.VMEM((4, CAP), jnp.int32),
        pltpu.VMEM((4, CAP), jnp.int32),
        pltpu.VMEM((2, CH_BIN), jnp.int32),
        pltpu.VMEM((2, CH_BIN), jnp.int32),
        pltpu.VMEM((16,), jnp.int32),
        pltpu.SemaphoreType.DMA,
        pltpu.SemaphoreType.DMA,
    ],
)


def _aggr_body(table, bsrc, bdst, bcnt, out, acc, sstage, dstage, cvec_buf,
               didx0, didx1, didx2, didx3,
               rows0, rows1, rows2, rows3,
               isem0, isem1, gsem0, gsem1, gsem2, gsem3,
               ssem0, ssem1, ssem2, ssem3):
  c = lax.axis_index("c")
  s = lax.axis_index("s")
  lane = lax.broadcasted_iota(jnp.int32, (16,), 0)
  didx = [didx0, didx1, didx2, didx3]
  rows = [rows0, rows1, rows2, rows3]
  isem = [isem0, isem1]
  gsem = [gsem0, gsem1, gsem2, gsem3]
  ssem = [ssem0, ssem1, ssem2, ssem3]

  def drain_idx(pb):
    pltpu.make_async_copy(bsrc.at[0, 0, pl.ds(0, SUP)], sstage.at[pb],
                          isem[0]).wait()
    pltpu.make_async_copy(bdst.at[0, 0, pl.ds(0, SUP)], dstage.at[pb],
                          isem[1]).wait()

  def drain_scat(b):
    pltpu.make_async_copy(table.at[pl.ds(0, KA)], rows[b], ssem[b]).wait()

  for p in range(2):
    q = c * 2 + p
    qbase = q * Q

    _zero_rows(rows0, KA)
    zbase = s * (ACC_ROWS // NS)
    for z in range(24):
      pltpu.sync_copy(rows0, acc.at[pl.ds(zbase + z * KA, KA)])
    pltpu.sync_copy(rows0.at[pl.ds(0, 40)], acc.at[pl.ds(zbase + 24 * KA, 40)])
    plsc.subcore_barrier()

    for k in range(2):
      t_idx = 2 * s + k

      pltpu.sync_copy(bcnt.at[t_idx], cvec_buf)
      v = cvec_buf[pl.ds(0, 16)]
      nsup = jnp.sum(jnp.where(lane == q, v, 0))

      def issue_idx(t, pb):
        pltpu.async_copy(bsrc.at[t_idx, q, pl.ds(t * SUP, SUP)],
                         sstage.at[pb], isem[0])
        pltpu.async_copy(bdst.at[t_idx, q, pl.ds(t * SUP, SUP)],
                         dstage.at[pb], isem[1])

      @pl.when(nsup > 0)
      def _():
        issue_idx(0, 0)

        def superstep(t, _):
          pb = t % 2
          drain_idx(pb)

          @pl.when(t < nsup - 1)
          def _():
            issue_idx(t + 1, 1 - pb)

          gcps = []
          for b in range(NB):
            @pl.when(t > 0)
            def _(b=b):
              drain_scat(b)
            for g in range(KA // 16):
              d = dstage[pb, pl.ds(b * KA + g * 16, 16)]
              didx[b][pl.ds(g * 16, 16)] = jnp.clip(d, 0, ACC_ROWS - 1)
            gcps.append(pltpu.async_copy(
                table.at[sstage.at[pb, pl.ds(b * KA, KA)]], rows[b], gsem[b]))
          for b in range(NB):
            gcps[b].wait()
            pltpu.async_copy(rows[b], acc.at[didx[b]], ssem[b], add=True)
          return 0

        lax.fori_loop(0, nsup, superstep, 0)
        for b in range(NB):
          drain_scat(b)

    plsc.subcore_barrier()

    obase = s * (Q // NS)
    for z in range(24):
      pltpu.sync_copy(acc.at[pl.ds(obase + z * KA, KA)],
                      out.at[pl.ds(qbase + obase + z * KA, KA)])
    pltpu.sync_copy(acc.at[pl.ds(obase + 24 * KA, 32)],
                    out.at[pl.ds(qbase + obase + 24 * KA, 32)])
    plsc.subcore_barrier()


_aggr = pl.kernel(
    _aggr_body,
    out_type=jax.ShapeDtypeStruct((N_PAD, H), jnp.float32),
    mesh=_mesh,
    compiler_params=_sc_params,
    scratch_types=[
        pltpu.VMEM_SHARED((ACC_ROWS, H), jnp.float32),
        pltpu.VMEM((2, SUP), jnp.int32),
        pltpu.VMEM((2, SUP), jnp.int32),
        pltpu.VMEM((16,), jnp.int32),
        pltpu.VMEM((KA,), jnp.int32),
        pltpu.VMEM((KA,), jnp.int32),
        pltpu.VMEM((KA,), jnp.int32),
        pltpu.VMEM((KA,), jnp.int32),
        pltpu.VMEM((KA, H), jnp.float32),
        pltpu.VMEM((KA, H), jnp.float32),
        pltpu.VMEM((KA, H), jnp.float32),
        pltpu.VMEM((KA, H), jnp.float32),
    ] + [pltpu.SemaphoreType.DMA] * 10,
)


CH_GB = 1024
NCH_GB = ET // CH_GB


def _gbmap_body(dstp, batchp, gb, bvm, dbuf, gbuf, isem, osem0, osem1):
  c = lax.axis_index("c")
  s = lax.axis_index("s")
  w = c * NS + s
  osem = [osem0, osem1]
  pltpu.sync_copy(batchp, bvm)

  def issue(t, pb):
    pltpu.async_copy(dstp.at[pl.ds(w * ET + t * CH_GB, CH_GB)],
                     dbuf.at[pb], isem)

  def drain_out(pb):
    pltpu.make_async_copy(dstp.at[pl.ds(0, CH_GB)], gbuf.at[0],
                          osem[pb]).wait()

  issue(0, 0)

  def chunk(t, _):
    pb = t % 2
    pltpu.make_async_copy(dstp.at[pl.ds(0, CH_GB)], dbuf.at[0], isem).wait()

    @pl.when(t < NCH_GB - 1)
    def _():
      issue(t + 1, 1 - pb)

    @pl.when((t > 1) & (pb == 0))
    def _():
      drain_out(0)

    @pl.when((t > 1) & (pb == 1))
    def _():
      drain_out(1)

    def grp(g, _):
      d = dbuf[pb, pl.ds(g * 16, 16)]
      gbuf[pb, pl.ds(g * 16, 16)] = plsc.load_gather(bvm, [d])
      return 0

    lax.fori_loop(0, CH_GB // 16, grp, 0)

    @pl.when(pb == 0)
    def _():
      pltpu.async_copy(gbuf.at[0], gb.at[pl.ds(w * ET + t * CH_GB, CH_GB)],
                       osem[0])

    @pl.when(pb == 1)
    def _():
      pltpu.async_copy(gbuf.at[1], gb.at[pl.ds(w * ET + t * CH_GB, CH_GB)],
                       osem[1])
    return 0

  lax.fori_loop(0, NCH_GB, chunk, 0)
  drain_out(0)
  drain_out(1)


_gbmap = pl.kernel(
    _gbmap_body,
    out_type=jax.ShapeDtypeStruct((E_PAD,), jnp.int32),
    mesh=_mesh,
    compiler_params=_sc_params,
    scratch_types=[
        pltpu.VMEM((BN_PAD,), jnp.int32),
        pltpu.VMEM((2, CH_GB), jnp.int32),
        pltpu.VMEM((2, CH_GB), jnp.int32),
        pltpu.SemaphoreType.DMA,
        pltpu.SemaphoreType.DMA,
        pltpu.SemaphoreType.DMA,
    ],
)


KP = 128
SUPP = NB * KP


def _pool_body(table3, h2, srcp, gbp, batchp,
               outp, outs, outc,
               accp, accs, accc, sstage, gstage,
               gbidx0, gbidx1, gbidx2, gbidx3,
               rows0, rows1, rows2, rows3, bidx,
               isem0, isem1, gsem0, gsem1, gsem2, gsem3,
               ssem0, ssem1, ssem2, ssem3):
  c = lax.axis_index("c")
  s = lax.axis_index("s")
  w = c * NS + s
  gbidx = [gbidx0, gbidx1, gbidx2, gbidx3]
  rows = [rows0, rows1, rows2, rows3]
  isem = [isem0, isem1]
  gsem = [gsem0, gsem1, gsem2, gsem3]
  ssem = [ssem0, ssem1, ssem2, ssem3]
  nsup = ET // SUPP

  _zero_rows(rows0, 34)
  gzb = s * (GP // NS)
  pltpu.sync_copy(rows0.at[pl.ds(0, 34)], accp.at[pl.ds(gzb, 34)])
  pltpu.sync_copy(rows0.at[pl.ds(0, 34)], accs.at[pl.ds(gzb, 34)])
  pltpu.sync_copy(rows0.at[pl.ds(0, 34)], accc.at[pl.ds(gzb, 34)])
  plsc.subcore_barrier()

  def drain_idx():
    pltpu.make_async_copy(srcp.at[pl.ds(0, SUPP)], sstage.at[0],
                          isem[0]).wait()
    pltpu.make_async_copy(gbp.at[pl.ds(0, SUPP)], gstage.at[0],
                          isem[1]).wait()

  def issue_idx(t, pb):
    eb = w * ET + t * SUPP
    pltpu.async_copy(srcp.at[pl.ds(eb, SUPP)], sstage.at[pb], isem[0])
    pltpu.async_copy(gbp.at[pl.ds(eb, SUPP)], gstage.at[pb], isem[1])

  def drain_scat(b):
    pltpu.make_async_copy(table3.at[pl.ds(0, KP)], rows[b], ssem[b]).wait()

  issue_idx(0, 0)

  def superstep(t, _):
    pb = t % 2
    drain_idx()

    @pl.when(t < nsup - 1)
    def _():
      issue_idx(t + 1, 1 - pb)

    gcps = []
    for b in range(NB):
      @pl.when(t > 0)
      def _(b=b):
        drain_scat(b)
      for g in range(KP // 16):
        d = gstage[pb, pl.ds(b * KP + g * 16, 16)]
        gbidx[b][pl.ds(g * 16, 16)] = jnp.clip(d, 0, GP - 1)
      gcps.append(pltpu.async_copy(
          table3.at[sstage.at[pb, pl.ds(b * KP, KP)]], rows[b], gsem[b]))
    for b in range(NB):
      gcps[b].wait()
      pltpu.async_copy(rows[b], accp.at[gbidx[b]], ssem[b], add=True)
    return 0

  lax.fori_loop(0, nsup, superstep, 0)
  for b in range(NB):
    drain_scat(b)

  def nchunk(j, _):
    nb = w * NODE_T + j * NODE_CH
    pltpu.sync_copy(batchp.at[pl.ds(nb, NODE_CH)], bidx)
    pltpu.sync_copy(h2.at[pl.ds(nb, NODE_CH)], rows0.at[pl.ds(0, NODE_CH)])
    pltpu.sync_copy(rows0.at[pl.ds(0, NODE_CH)], accs.at[bidx], add=True)
    return 0

  lax.fori_loop(0, NODE_NCH, nchunk, 0)

  _fill_ones(rows0, NODE_CH)

  def cchunk(j, _):
    nb = w * NODE_T + j * NODE_CH
    pltpu.sync_copy(batchp.at[pl.ds(nb, NODE_CH)], bidx)
    pltpu.sync_copy(rows0.at[pl.ds(0, NODE_CH)], accc.at[bidx], add=True)
    return 0

  lax.fori_loop(0, NODE_NCH, cchunk, 0)
  plsc.subcore_barrier()

  ob = s * (G // NS)
  pltpu.sync_copy(accp.at[pl.ds(ob, G // NS)], outp.at[c, pl.ds(ob, G // NS)])
  pltpu.sync_copy(accs.at[pl.ds(ob, G // NS)], outs.at[c, pl.ds(ob, G // NS)])
  pltpu.sync_copy(accc.at[pl.ds(ob, G // NS)], outc.at[c, pl.ds(ob, G // NS)])


_pool = pl.kernel(
    _pool_body,
    out_type=[
        jax.ShapeDtypeStruct((NC, G, H), jnp.float32),
        jax.ShapeDtypeStruct((NC, G, H), jnp.float32),
        jax.ShapeDtypeStruct((NC, G, H), jnp.float32),
    ],
    mesh=_mesh,
    compiler_params=_sc_params,
    scratch_types=[
        pltpu.VMEM_SHARED((GP, H), jnp.float32),
        pltpu.VMEM_SHARED((GP, H), jnp.float32),
        pltpu.VMEM_SHARED((GP, H), jnp.float32),
        pltpu.VMEM((2, SUPP), jnp.int32),
        pltpu.VMEM((2, SUPP), jnp.int32),
        pltpu.VMEM((KP,), jnp.int32),
        pltpu.VMEM((KP,), jnp.int32),
        pltpu.VMEM((KP,), jnp.int32),
        pltpu.VMEM((KP,), jnp.int32),
        pltpu.VMEM((KP, H), jnp.float32),
        pltpu.VMEM((KP, H), jnp.float32),
        pltpu.VMEM((KP, H), jnp.float32),
        pltpu.VMEM((KP, H), jnp.float32),
        pltpu.VMEM((NODE_CH,), jnp.int32),
    ] + [pltpu.SemaphoreType.DMA] * 10,
)


_BLK = 1024
_NBLK = N_PAD // _BLK


def _dense0_body(x, w1rel, w1root, b1, t1, r1):
  t1[:] = lax.dot_general(x[:], w1rel[:], (((1,), (1,)), ((), ())),
                          preferred_element_type=jnp.float32)
  r1[:] = b1[:] + lax.dot_general(x[:], w1root[:], (((1,), (1,)), ((), ())),
                                  preferred_element_type=jnp.float32)


def _dense0(x_p, w1rel, w1root, b1):
  return pl.pallas_call(
      _dense0_body,
      grid=(_NBLK,),
      in_specs=[
          pl.BlockSpec((_BLK, IN), lambda i: (i, 0)),
          pl.BlockSpec((H, IN), lambda i: (0, 0)),
          pl.BlockSpec((H, IN), lambda i: (0, 0)),
          pl.BlockSpec((1, H), lambda i: (0, 0)),
      ],
      out_specs=[
          pl.BlockSpec((_BLK, H), lambda i: (i, 0)),
          pl.BlockSpec((_BLK, H), lambda i: (i, 0)),
      ],
      out_shape=[
          jax.ShapeDtypeStruct((N_PAD, H), jnp.float32),
          jax.ShapeDtypeStruct((N_PAD, H), jnp.float32),
      ],
  )(x_p, w1rel, w1root, b1)


def _dense_mid1_body(a, r, wrel_next, wroot_next, b_next, t_o, r_o):
  h = jnp.maximum(a[:] + r[:], 0.0)
  t_o[:] = lax.dot_general(h, wrel_next[:], (((1,), (1,)), ((), ())),
                           preferred_element_type=jnp.float32)
  r_o[:] = b_next[:] + lax.dot_general(h, wroot_next[:],
                                       (((1,), (1,)), ((), ())),
                                       preferred_element_type=jnp.float32)


def _dense_mid1(a, r, wrel_next, wroot_next, b_next):
  return pl.pallas_call(
      _dense_mid1_body,
      grid=(_NBLK,),
      in_specs=[
          pl.BlockSpec((_BLK, H), lambda i: (i, 0)),
          pl.BlockSpec((_BLK, H), lambda i: (i, 0)),
          pl.BlockSpec((H, H), lambda i: (0, 0)),
          pl.BlockSpec((H, H), lambda i: (0, 0)),
          pl.BlockSpec((1, H), lambda i: (0, 0)),
      ],
      out_specs=[
          pl.BlockSpec((_BLK, H), lambda i: (i, 0)),
          pl.BlockSpec((_BLK, H), lambda i: (i, 0)),
      ],
      out_shape=[
          jax.ShapeDtypeStruct((N_PAD, H), jnp.float32),
          jax.ShapeDtypeStruct((N_PAD, H), jnp.float32),
      ],
  )(a, r, wrel_next, wroot_next, b_next)


def _dense_mid2_body(a, r, wrel_next, t_o, h_o):
  h = jnp.maximum(a[:] + r[:], 0.0)
  h_o[:] = h
  t_o[:] = lax.dot_general(h, wrel_next[:], (((1,), (1,)), ((), ())),
                           preferred_element_type=jnp.float32)


def _dense_mid2(a, r, wrel_next):
  return pl.pallas_call(
      _dense_mid2_body,
      grid=(_NBLK,),
      in_specs=[
          pl.BlockSpec((_BLK, H), lambda i: (i, 0)),
          pl.BlockSpec((_BLK, H), lambda i: (i, 0)),
          pl.BlockSpec((H, H), lambda i: (0, 0)),
      ],
      out_specs=[
          pl.BlockSpec((_BLK, H), lambda i: (i, 0)),
          pl.BlockSpec((_BLK, H), lambda i: (i, 0)),
      ],
      out_shape=[
          jax.ShapeDtypeStruct((N_PAD, H), jnp.float32),
          jax.ShapeDtypeStruct((N_PAD, H), jnp.float32),
      ],
  )(a, r, wrel_next)


def _final_body(p3p, s2p, cntp, b3, w3root, wlin, blin, out):
  p3 = p3p[0] + p3p[1]
  s2 = s2p[0] + s2p[1]
  cnt = cntp[0, :, 0:1] + cntp[1, :, 0:1]
  cntc = jnp.maximum(cnt, 1.0)
  pooled = p3 / cntc + b3[:] + lax.dot_general(
      s2 / cntc, w3root[:], (((1,), (1,)), ((), ())),
      preferred_element_type=jnp.float32)
  pooled = jnp.where(cnt > 0.5, pooled, 0.0)
  out[:] = lax.dot_general(pooled, wlin[:], (((1,), (1,)), ((), ())),
                           preferred_element_type=jnp.float32) + blin[:]


def _final(p3p, s2p, cntp, b3, w3root, wlin, blin):
  return pl.pallas_call(
      _final_body,
      out_shape=jax.ShapeDtypeStruct((G, OUT), jnp.float32),
  )(p3p, s2p, cntp, b3, w3root, wlin, blin)


def kernel(x, edge_index, batch, W1_rel, b1_rel, W1_root, W2_rel, b2_rel,
           W2_root, W3_rel, b3_rel, W3_root, W_lin, b_lin):
  src = edge_index[0].astype(jnp.int32)
  dst = edge_index[1].astype(jnp.int32)
  batch32 = batch.astype(jnp.int32)

  srcp = jnp.pad(src, (0, E_PAD - E))
  dstp = jnp.pad(dst, (0, E_PAD - E), constant_values=N_PAD)
  batchp = jnp.pad(batch32, (0, BN_PAD - N), constant_values=G)

  x_p = jnp.pad(x, ((0, N_PAD - N), (0, 0)))

  b1 = b1_rel.reshape(1, H)
  b2 = b2_rel.reshape(1, H)
  b3 = b3_rel.reshape(1, H)
  blin = b_lin.reshape(1, OUT)

  bsrc, bdst, bcnt = _bin(srcp, dstp)
  table1, r1 = _dense0(x_p, W1_rel, W1_root, b1)
  a1 = _aggr(table1, bsrc, bdst, bcnt)
  table2, r2 = _dense_mid1(a1, r1, W2_rel, W2_root, b2)
  a2 = _aggr(table2, bsrc, bdst, bcnt)
  table3, h2 = _dense_mid2(a2, r2, W3_rel)
  gb = _gbmap(dstp, batchp)
  p3p, s2p, cntp = _pool(table3, h2, srcp, gb, batchp)
  return _final(p3p, s2p, cntp, b3, W3_root, W_lin, blin)


# speedup vs baseline: 1.1120x; 1.1120x over previous
import functools

import jax
import jax.numpy as jnp
from jax import lax
from jax.experimental import pallas as pl
from jax.experimental.pallas import tpu as pltpu
from jax.experimental.pallas import tpu_sc as plsc

N = 100000
E = 1600000
G = 512
IN = 50
H = 64
OUT = 4

NC = 2
NS = 16
NW = NC * NS

Q = 25088
N_PAD = 4 * Q
ACC_ROWS = Q + 128
K = 128
KA = 128
ET = 50176
E_PAD = ET * NW
NCHUNK = ET // K

GP = 544
BN_PAD = N_PAD + 16
NODE_CH = 64
NODE_T = N_PAD // NW
NODE_NCH = NODE_T // NODE_CH

_mesh = plsc.VectorSubcoreMesh(
    core_axis_name="c", subcore_axis_name="s", num_cores=NC, num_subcores=NS)
_sc_params = pltpu.CompilerParams(use_tc_tiling_on_sc=False,
                                  needs_layout_passes=False)


def _zero_rows(rows, nrows):
  zero = jnp.zeros((16,), jnp.float32)

  def body(i, _):
    r = i // 4
    l = i % 4
    rows[r, pl.ds(l * 16, 16)] = zero
    return 0

  lax.fori_loop(0, nrows * 4, body, 0)


def _fill_ones(rows, nrows):
  one = jnp.ones((16,), jnp.float32)

  def body(i, _):
    r = i // 4
    l = i % 4
    rows[r, pl.ds(l * 16, 16)] = one
    return 0

  lax.fori_loop(0, nrows * 4, body, 0)


NB = 3
SUP = NB * KA

CAP = 14336
CH_BIN = 1792
NCH_BIN = ET // CH_BIN


def _bin_body(srcp, dstp, bsrc, bdst, bcnt,
              stage_s, stage_d, sbuf, dbuf, cntv, isem0, isem1):
  c = lax.axis_index("c")
  s = lax.axis_index("s")
  w = c * NS + s
  lane = lax.broadcasted_iota(jnp.int32, (16,), 0)

  def issue_idx(t, pb):
    eb = w * ET + t * CH_BIN
    pltpu.async_copy(srcp.at[pl.ds(eb, CH_BIN)], sbuf.at[pb], isem0)
    pltpu.async_copy(dstp.at[pl.ds(eb, CH_BIN)], dbuf.at[pb], isem1)

  def drain_idx():
    pltpu.make_async_copy(srcp.at[pl.ds(0, CH_BIN)], sbuf.at[0], isem0).wait()
    pltpu.make_async_copy(dstp.at[pl.ds(0, CH_BIN)], dbuf.at[0], isem1).wait()

  issue_idx(0, 0)

  def chunk(t, cnts):
    pb = t % 2
    drain_idx()

    @pl.when(t < NCH_BIN - 1)
    def _():
      issue_idx(t + 1, 1 - pb)

    def grp(g, cnts):
      d = dbuf[pb, pl.ds(g * 16, 16)]
      sv = sbuf[pb, pl.ds(g * 16, 16)]
      qv = ((d >= Q).astype(jnp.int32) + (d >= 2 * Q).astype(jnp.int32)
            + (d >= 3 * Q).astype(jnp.int32))
      rel = d - qv * Q
      new = []
      for b in range(4):
        cb = cnts[b]
        keep = (qv == b) & (rel < Q)
        mi = keep.astype(jnp.int32)
        m = keep & (cb < CAP - 16)
        pos = cb + plsc.cumsum(mi) - mi
        plsc.store_scatter(stage_s.at[b], [pos], sv, mask=m)
        plsc.store_scatter(stage_d.at[b], [pos], rel, mask=m)
        new.append(cb + jnp.sum(mi))
      return tuple(new)

    return lax.fori_loop(0, CH_BIN // 16, grp, cnts)

  cnts = lax.fori_loop(0, NCH_BIN, chunk, (0, 0, 0, 0))

  nchs = []
  for b in range(4):
    cb = jnp.minimum(cnts[b], CAP - SUP)
    target = ((cb + SUP - 1) // SUP) * SUP
    npadg = (target - cb + 15) // 16

    def padg(i, _, b=b, cb=cb):
      vsrc = (w * 1237 + i * 577 + lane * 61) % N
      vdst = Q + ((i * 16 + lane) & 127)
      stage_s[b, pl.ds(cb + i * 16, 16)] = vsrc
      stage_d[b, pl.ds(cb + i * 16, 16)] = vdst
      return 0

    lax.fori_loop(0, npadg, padg, 0)
    nchs.append(target // SUP)
    nfl = (target + 511) // 512

    def flush(j, _, b=b):
      pltpu.sync_copy(stage_s.at[b, pl.ds(j * 512, 512)],
                      bsrc.at[w, b, pl.ds(j * 512, 512)])
      pltpu.sync_copy(stage_d.at[b, pl.ds(j * 512, 512)],
                      bdst.at[w, b, pl.ds(j * 512, 512)])
      return 0

    lax.fori_loop(0, nfl, flush, 0)

  cvec = jnp.where(lane == 0, nchs[0],
                   jnp.where(lane == 1, nchs[1],
                             jnp.where(lane == 2, nchs[2],
                                       jnp.where(lane == 3, nchs[3], 0))))
  cntv[pl.ds(0, 16)] = cvec
  pltpu.sync_copy(cntv, bcnt.at[w])


_bin = pl.kernel(
    _bin_body,
    out_type=[
        jax.ShapeDtypeStruct((NW, 4, CAP), jnp.int32),
        jax.ShapeDtypeStruct((NW, 4, CAP), jnp.int32),
        jax.ShapeDtypeStruct((NW, 16), jnp.int32),
    ],
    mesh=_mesh,
    compiler_params=_sc_params,
    scratch_types=[
        pltpu.VMEM((4, CAP), jnp.int32),
        pltpu.VMEM((4, CAP), jnp.int32),
        pltpu.VMEM((2, CH_BIN), jnp.int32),
        pltpu.VMEM((2, CH_BIN), jnp.int32),
        pltpu.VMEM((16,), jnp.int32),
        pltpu.SemaphoreType.DMA,
        pltpu.SemaphoreType.DMA,
    ],
)


def _aggr_body(table, bsrc, bdst, bcnt, out, acc, sstage, dstage, cvec_buf,
               didx0, didx1, didx2,
               rows0, rows1, rows2,
               isem0, isem1, gsem0, gsem1, gsem2,
               ssem0, ssem1, ssem2):
  c = lax.axis_index("c")
  s = lax.axis_index("s")
  lane = lax.broadcasted_iota(jnp.int32, (16,), 0)
  didx = [didx0, didx1, didx2]
  rows = [rows0, rows1, rows2]
  isem = [isem0, isem1]
  gsem = [gsem0, gsem1, gsem2]
  ssem = [ssem0, ssem1, ssem2]

  def drain_idx(pb):
    pltpu.make_async_copy(bsrc.at[0, 0, pl.ds(0, SUP)], sstage.at[pb],
                          isem[0]).wait()
    pltpu.make_async_copy(bdst.at[0, 0, pl.ds(0, SUP)], dstage.at[pb],
                          isem[1]).wait()

  def drain_scat(b):
    pltpu.make_async_copy(table.at[pl.ds(0, KA)], rows[b], ssem[b]).wait()

  for p in range(2):
    q = c * 2 + p
    qbase = q * Q

    _zero_rows(rows0, KA)
    zbase = s * (ACC_ROWS // NS)
    zfull, zrem = (ACC_ROWS // NS) // KA, (ACC_ROWS // NS) % KA
    for z in range(zfull):
      pltpu.sync_copy(rows0, acc.at[pl.ds(zbase + z * KA, KA)])
    if zrem:
      pltpu.sync_copy(rows0.at[pl.ds(0, zrem)],
                      acc.at[pl.ds(zbase + zfull * KA, zrem)])
    plsc.subcore_barrier()

    for k in range(2):
      t_idx = 2 * s + k

      pltpu.sync_copy(bcnt.at[t_idx], cvec_buf)
      v = cvec_buf[pl.ds(0, 16)]
      nsup = jnp.sum(jnp.where(lane == q, v, 0))

      def issue_idx(t, pb):
        pltpu.async_copy(bsrc.at[t_idx, q, pl.ds(t * SUP, SUP)],
                         sstage.at[pb], isem[0])
        pltpu.async_copy(bdst.at[t_idx, q, pl.ds(t * SUP, SUP)],
                         dstage.at[pb], isem[1])

      @pl.when(nsup > 0)
      def _():
        issue_idx(0, 0)

        def superstep(t, _):
          pb = t % 2
          drain_idx(pb)

          @pl.when(t < nsup - 1)
          def _():
            issue_idx(t + 1, 1 - pb)

          gcps = []
          for b in range(NB):
            @pl.when(t > 0)
            def _(b=b):
              drain_scat(b)
            for g in range(KA // 16):
              d = dstage[pb, pl.ds(b * KA + g * 16, 16)]
              didx[b][pl.ds(g * 16, 16)] = jnp.clip(d, 0, ACC_ROWS - 1)
            gcps.append(pltpu.async_copy(
                table.at[sstage.at[pb, pl.ds(b * KA, KA)]], rows[b], gsem[b]))
          for b in range(NB):
            gcps[b].wait()
            pltpu.async_copy(rows[b], acc.at[didx[b]], ssem[b], add=True)
          return 0

        lax.fori_loop(0, nsup, superstep, 0)
        for b in range(NB):
          drain_scat(b)

    plsc.subcore_barrier()

    obase = s * (Q // NS)
    ofull, orem = (Q // NS) // KA, (Q // NS) % KA
    for z in range(ofull):
      pltpu.sync_copy(acc.at[pl.ds(obase + z * KA, KA)],
                      out.at[pl.ds(qbase + obase + z * KA, KA)])
    if orem:
      pltpu.sync_copy(acc.at[pl.ds(obase + ofull * KA, orem)],
                      out.at[pl.ds(qbase + obase + ofull * KA, orem)])
    plsc.subcore_barrier()


_aggr = pl.kernel(
    _aggr_body,
    out_type=jax.ShapeDtypeStruct((N_PAD, H), jnp.float32),
    mesh=_mesh,
    compiler_params=_sc_params,
    scratch_types=[
        pltpu.VMEM_SHARED((ACC_ROWS, H), jnp.float32),
        pltpu.VMEM((2, SUP), jnp.int32),
        pltpu.VMEM((2, SUP), jnp.int32),
        pltpu.VMEM((16,), jnp.int32),
        pltpu.VMEM((KA,), jnp.int32),
        pltpu.VMEM((KA,), jnp.int32),
        pltpu.VMEM((KA,), jnp.int32),
        pltpu.VMEM((KA, H), jnp.float32),
        pltpu.VMEM((KA, H), jnp.float32),
        pltpu.VMEM((KA, H), jnp.float32),
    ] + [pltpu.SemaphoreType.DMA] * 8,
)


CH_GB = 1024
NCH_GB = ET // CH_GB


def _gbmap_body(dstp, batchp, gb, bvm, dbuf, gbuf, isem, osem0, osem1):
  c = lax.axis_index("c")
  s = lax.axis_index("s")
  w = c * NS + s
  osem = [osem0, osem1]
  pltpu.sync_copy(batchp, bvm)

  def issue(t, pb):
    pltpu.async_copy(dstp.at[pl.ds(w * ET + t * CH_GB, CH_GB)],
                     dbuf.at[pb], isem)

  def drain_out(pb):
    pltpu.make_async_copy(dstp.at[pl.ds(0, CH_GB)], gbuf.at[0],
                          osem[pb]).wait()

  issue(0, 0)

  def chunk(t, _):
    pb = t % 2
    pltpu.make_async_copy(dstp.at[pl.ds(0, CH_GB)], dbuf.at[0], isem).wait()

    @pl.when(t < NCH_GB - 1)
    def _():
      issue(t + 1, 1 - pb)

    @pl.when((t > 1) & (pb == 0))
    def _():
      drain_out(0)

    @pl.when((t > 1) & (pb == 1))
    def _():
      drain_out(1)

    def grp(g, _):
      d = dbuf[pb, pl.ds(g * 16, 16)]
      gbuf[pb, pl.ds(g * 16, 16)] = plsc.load_gather(bvm, [d])
      return 0

    lax.fori_loop(0, CH_GB // 16, grp, 0)

    @pl.when(pb == 0)
    def _():
      pltpu.async_copy(gbuf.at[0], gb.at[pl.ds(w * ET + t * CH_GB, CH_GB)],
                       osem[0])

    @pl.when(pb == 1)
    def _():
      pltpu.async_copy(gbuf.at[1], gb.at[pl.ds(w * ET + t * CH_GB, CH_GB)],
                       osem[1])
    return 0

  lax.fori_loop(0, NCH_GB, chunk, 0)
  drain_out(0)
  drain_out(1)


_gbmap = pl.kernel(
    _gbmap_body,
    out_type=jax.ShapeDtypeStruct((E_PAD,), jnp.int32),
    mesh=_mesh,
    compiler_params=_sc_params,
    scratch_types=[
        pltpu.VMEM((BN_PAD,), jnp.int32),
        pltpu.VMEM((2, CH_GB), jnp.int32),
        pltpu.VMEM((2, CH_GB), jnp.int32),
        pltpu.SemaphoreType.DMA,
        pltpu.SemaphoreType.DMA,
        pltpu.SemaphoreType.DMA,
    ],
)


KP = 128
NBP = 4
SUPP = NBP * KP


def _pool_body(table3, h2, srcp, gbp, batchp,
               outp, outs, outc,
               accp, accs, accc, sstage, gstage,
               gbidx0, gbidx1, gbidx2, gbidx3,
               rows0, rows1, rows2, rows3, bidx,
               isem0, isem1, gsem0, gsem1, gsem2, gsem3,
               ssem0, ssem1, ssem2, ssem3):
  c = lax.axis_index("c")
  s = lax.axis_index("s")
  w = c * NS + s
  gbidx = [gbidx0, gbidx1, gbidx2, gbidx3]
  rows = [rows0, rows1, rows2, rows3]
  isem = [isem0, isem1]
  gsem = [gsem0, gsem1, gsem2, gsem3]
  ssem = [ssem0, ssem1, ssem2, ssem3]
  nsup = ET // SUPP

  _zero_rows(rows0, 34)
  gzb = s * (GP // NS)
  pltpu.sync_copy(rows0.at[pl.ds(0, 34)], accp.at[pl.ds(gzb, 34)])
  pltpu.sync_copy(rows0.at[pl.ds(0, 34)], accs.at[pl.ds(gzb, 34)])
  pltpu.sync_copy(rows0.at[pl.ds(0, 34)], accc.at[pl.ds(gzb, 34)])
  plsc.subcore_barrier()

  def drain_idx():
    pltpu.make_async_copy(srcp.at[pl.ds(0, SUPP)], sstage.at[0],
                          isem[0]).wait()
    pltpu.make_async_copy(gbp.at[pl.ds(0, SUPP)], gstage.at[0],
                          isem[1]).wait()

  def issue_idx(t, pb):
    eb = w * ET + t * SUPP
    pltpu.async_copy(srcp.at[pl.ds(eb, SUPP)], sstage.at[pb], isem[0])
    pltpu.async_copy(gbp.at[pl.ds(eb, SUPP)], gstage.at[pb], isem[1])

  def drain_scat(b):
    pltpu.make_async_copy(table3.at[pl.ds(0, KP)], rows[b], ssem[b]).wait()

  issue_idx(0, 0)

  def superstep(t, _):
    pb = t % 2
    drain_idx()

    @pl.when(t < nsup - 1)
    def _():
      issue_idx(t + 1, 1 - pb)

    gcps = []
    for b in range(NBP):
      @pl.when(t > 0)
      def _(b=b):
        drain_scat(b)
      for g in range(KP // 16):
        d = gstage[pb, pl.ds(b * KP + g * 16, 16)]
        gbidx[b][pl.ds(g * 16, 16)] = jnp.clip(d, 0, GP - 1)
      gcps.append(pltpu.async_copy(
          table3.at[sstage.at[pb, pl.ds(b * KP, KP)]], rows[b], gsem[b]))
    for b in range(NBP):
      gcps[b].wait()
      pltpu.async_copy(rows[b], accp.at[gbidx[b]], ssem[b], add=True)
    return 0

  lax.fori_loop(0, nsup, superstep, 0)
  for b in range(NBP):
    drain_scat(b)

  def nchunk(j, _):
    nb = w * NODE_T + j * NODE_CH
    pltpu.sync_copy(batchp.at[pl.ds(nb, NODE_CH)], bidx)
    pltpu.sync_copy(h2.at[pl.ds(nb, NODE_CH)], rows0.at[pl.ds(0, NODE_CH)])
    pltpu.sync_copy(rows0.at[pl.ds(0, NODE_CH)], accs.at[bidx], add=True)
    return 0

  lax.fori_loop(0, NODE_NCH, nchunk, 0)

  _fill_ones(rows0, NODE_CH)

  def cchunk(j, _):
    nb = w * NODE_T + j * NODE_CH
    pltpu.sync_copy(batchp.at[pl.ds(nb, NODE_CH)], bidx)
    pltpu.sync_copy(rows0.at[pl.ds(0, NODE_CH)], accc.at[bidx], add=True)
    return 0

  lax.fori_loop(0, NODE_NCH, cchunk, 0)
  plsc.subcore_barrier()

  ob = s * (G // NS)
  pltpu.sync_copy(accp.at[pl.ds(ob, G // NS)], outp.at[c, pl.ds(ob, G // NS)])
  pltpu.sync_copy(accs.at[pl.ds(ob, G // NS)], outs.at[c, pl.ds(ob, G // NS)])
  pltpu.sync_copy(accc.at[pl.ds(ob, G // NS)], outc.at[c, pl.ds(ob, G // NS)])


_pool = pl.kernel(
    _pool_body,
    out_type=[
        jax.ShapeDtypeStruct((NC, G, H), jnp.float32),
        jax.ShapeDtypeStruct((NC, G, H), jnp.float32),
        jax.ShapeDtypeStruct((NC, G, H), jnp.float32),
    ],
    mesh=_mesh,
    compiler_params=_sc_params,
    scratch_types=[
        pltpu.VMEM_SHARED((GP, H), jnp.float32),
        pltpu.VMEM_SHARED((GP, H), jnp.float32),
        pltpu.VMEM_SHARED((GP, H), jnp.float32),
        pltpu.VMEM((2, SUPP), jnp.int32),
        pltpu.VMEM((2, SUPP), jnp.int32),
        pltpu.VMEM((KP,), jnp.int32),
        pltpu.VMEM((KP,), jnp.int32),
        pltpu.VMEM((KP,), jnp.int32),
        pltpu.VMEM((KP,), jnp.int32),
        pltpu.VMEM((KP, H), jnp.float32),
        pltpu.VMEM((KP, H), jnp.float32),
        pltpu.VMEM((KP, H), jnp.float32),
        pltpu.VMEM((KP, H), jnp.float32),
        pltpu.VMEM((NODE_CH,), jnp.int32),
    ] + [pltpu.SemaphoreType.DMA] * 10,
)


_BLK = 1024
_NBLK = N_PAD // _BLK


def _dense0_body(x, w1rel, w1root, b1, t1, r1):
  t1[:] = lax.dot_general(x[:], w1rel[:], (((1,), (1,)), ((), ())),
                          preferred_element_type=jnp.float32)
  r1[:] = b1[:] + lax.dot_general(x[:], w1root[:], (((1,), (1,)), ((), ())),
                                  preferred_element_type=jnp.float32)


def _dense0(x_p, w1rel, w1root, b1):
  return pl.pallas_call(
      _dense0_body,
      grid=(_NBLK,),
      in_specs=[
          pl.BlockSpec((_BLK, IN), lambda i: (i, 0)),
          pl.BlockSpec((H, IN), lambda i: (0, 0)),
          pl.BlockSpec((H, IN), lambda i: (0, 0)),
          pl.BlockSpec((1, H), lambda i: (0, 0)),
      ],
      out_specs=[
          pl.BlockSpec((_BLK, H), lambda i: (i, 0)),
          pl.BlockSpec((_BLK, H), lambda i: (i, 0)),
      ],
      out_shape=[
          jax.ShapeDtypeStruct((N_PAD, H), jnp.float32),
          jax.ShapeDtypeStruct((N_PAD, H), jnp.float32),
      ],
  )(x_p, w1rel, w1root, b1)


def _dense_mid1_body(a, r, wrel_next, wroot_next, b_next, t_o, r_o):
  h = jnp.maximum(a[:] + r[:], 0.0)
  t_o[:] = lax.dot_general(h, wrel_next[:], (((1,), (1,)), ((), ())),
                           preferred_element_type=jnp.float32)
  r_o[:] = b_next[:] + lax.dot_general(h, wroot_next[:],
                                       (((1,), (1,)), ((), ())),
                                       preferred_element_type=jnp.float32)


def _dense_mid1(a, r, wrel_next, wroot_next, b_next):
  return pl.pallas_call(
      _dense_mid1_body,
      grid=(_NBLK,),
      in_specs=[
          pl.BlockSpec((_BLK, H), lambda i: (i, 0)),
          pl.BlockSpec((_BLK, H), lambda i: (i, 0)),
          pl.BlockSpec((H, H), lambda i: (0, 0)),
          pl.BlockSpec((H, H), lambda i: (0, 0)),
          pl.BlockSpec((1, H), lambda i: (0, 0)),
      ],
      out_specs=[
          pl.BlockSpec((_BLK, H), lambda i: (i, 0)),
          pl.BlockSpec((_BLK, H), lambda i: (i, 0)),
      ],
      out_shape=[
          jax.ShapeDtypeStruct((N_PAD, H), jnp.float32),
          jax.ShapeDtypeStruct((N_PAD, H), jnp.float32),
      ],
  )(a, r, wrel_next, wroot_next, b_next)


def _dense_mid2_body(a, r, wrel_next, t_o, h_o):
  h = jnp.maximum(a[:] + r[:], 0.0)
  h_o[:] = h
  t_o[:] = lax.dot_general(h, wrel_next[:], (((1,), (1,)), ((), ())),
                           preferred_element_type=jnp.float32)


def _dense_mid2(a, r, wrel_next):
  return pl.pallas_call(
      _dense_mid2_body,
      grid=(_NBLK,),
      in_specs=[
          pl.BlockSpec((_BLK, H), lambda i: (i, 0)),
          pl.BlockSpec((_BLK, H), lambda i: (i, 0)),
          pl.BlockSpec((H, H), lambda i: (0, 0)),
      ],
      out_specs=[
          pl.BlockSpec((_BLK, H), lambda i: (i, 0)),
          pl.BlockSpec((_BLK, H), lambda i: (i, 0)),
      ],
      out_shape=[
          jax.ShapeDtypeStruct((N_PAD, H), jnp.float32),
          jax.ShapeDtypeStruct((N_PAD, H), jnp.float32),
      ],
  )(a, r, wrel_next)


def _final_body(p3p, s2p, cntp, b3, w3root, wlin, blin, out):
  p3 = p3p[0] + p3p[1]
  s2 = s2p[0] + s2p[1]
  cnt = cntp[0, :, 0:1] + cntp[1, :, 0:1]
  cntc = jnp.maximum(cnt, 1.0)
  pooled = p3 / cntc + b3[:] + lax.dot_general(
      s2 / cntc, w3root[:], (((1,), (1,)), ((), ())),
      preferred_element_type=jnp.float32)
  pooled = jnp.where(cnt > 0.5, pooled, 0.0)
  out[:] = lax.dot_general(pooled, wlin[:], (((1,), (1,)), ((), ())),
                           preferred_element_type=jnp.float32) + blin[:]


def _final(p3p, s2p, cntp, b3, w3root, wlin, blin):
  return pl.pallas_call(
      _final_body,
      out_shape=jax.ShapeDtypeStruct((G, OUT), jnp.float32),
  )(p3p, s2p, cntp, b3, w3root, wlin, blin)


def kernel(x, edge_index, batch, W1_rel, b1_rel, W1_root, W2_rel, b2_rel,
           W2_root, W3_rel, b3_rel, W3_root, W_lin, b_lin):
  src = edge_index[0].astype(jnp.int32)
  dst = edge_index[1].astype(jnp.int32)
  batch32 = batch.astype(jnp.int32)

  npad = E_PAD - E
  pad_i = jnp.arange(npad, dtype=jnp.int32)
  srcp = jnp.concatenate([src, (pad_i * 61) % N])
  dstp = jnp.concatenate([dst, N_PAD + (pad_i % 16)])
  bp_i = jnp.arange(BN_PAD - N, dtype=jnp.int32)
  batchp = jnp.concatenate([batch32, G + (bp_i % 16)])

  x_p = jnp.pad(x, ((0, N_PAD - N), (0, 0)))

  b1 = b1_rel.reshape(1, H)
  b2 = b2_rel.reshape(1, H)
  b3 = b3_rel.reshape(1, H)
  blin = b_lin.reshape(1, OUT)

  bsrc, bdst, bcnt = _bin(srcp, dstp)
  table1, r1 = _dense0(x_p, W1_rel, W1_root, b1)
  a1 = _aggr(table1, bsrc, bdst, bcnt)
  table2, r2 = _dense_mid1(a1, r1, W2_rel, W2_root, b2)
  a2 = _aggr(table2, bsrc, bdst, bcnt)
  table3, h2 = _dense_mid2(a2, r2, W3_rel)
  gb = _gbmap(dstp, batchp)
  p3p, s2p, cntp = _pool(table3, h2, srcp, gb, batchp)
  return _final(p3p, s2p, cntp, b3, W3_root, W_lin, blin)
